# Initial kernel scaffold; baseline (speedup 1.0000x reference)
import jax
import jax.numpy as jnp
from jax.experimental import pallas as pl


def kernel(x, t, pos, edge_index, edge_attr, batch, atom_emb, bond_emb, W_t1, b_t1, W_t2, b_t2, W_msg, b_msg, W_upd, b_upd, W_v, gamma, beta, W_down, b_down):
    raise NotImplementedError("write your pallas kernel here")


# speedup vs baseline: 6.2849x; 6.2849x over previous
import functools

import jax
import jax.numpy as jnp
from jax.experimental import pallas as pl
from jax.experimental.pallas import tpu as pltpu

SDIM = 64
VDIM = 16
V3 = 3 * VDIM
MDIM = SDIM + 2 * VDIM
BN = 512


def _silu(x):
    return x * jax.nn.sigmoid(x)


def _node_pre_body(s_ref, v_ref, gamma_ref, beta_ref, w1_ref, w2_ref,
                   sbar_ref, vbar_ref, a_ref, b2_ref):
    s = s_ref[...]
    mu = jnp.mean(s, axis=-1, keepdims=True)
    var = jnp.mean((s - mu) * (s - mu), axis=-1, keepdims=True)
    sbar = (s - mu) * jax.lax.rsqrt(var + 1e-5) * gamma_ref[...] + beta_ref[...]
    sbar_ref[...] = sbar
    v = v_ref[...]
    inv = jax.lax.rsqrt(jnp.sum(v * v, axis=-1, keepdims=True) / VDIM + 1e-5)
    vbar_ref[...] = v * inv
    a_ref[...] = jnp.dot(sbar, w1_ref[...], preferred_element_type=jnp.float32)
    b2_ref[...] = jnp.dot(sbar, w2_ref[...], preferred_element_type=jnp.float32)


def _node_pre(s, vflat, gamma_i, beta_i, w1, w2):
    n = s.shape[0]
    grid = (pl.cdiv(n, BN),)
    out_shapes = (
        jax.ShapeDtypeStruct((n, SDIM), jnp.float32),
        jax.ShapeDtypeStruct((n, V3), jnp.float32),
        jax.ShapeDtypeStruct((n, MDIM), jnp.float32),
        jax.ShapeDtypeStruct((n, MDIM), jnp.float32),
    )
    return pl.pallas_call(
        _node_pre_body,
        grid=grid,
        in_specs=[
            pl.BlockSpec((BN, SDIM), lambda i: (i, 0)),
            pl.BlockSpec((BN, V3), lambda i: (i, 0)),
            pl.BlockSpec((1, SDIM), lambda i: (0, 0)),
            pl.BlockSpec((1, SDIM), lambda i: (0, 0)),
            pl.BlockSpec((SDIM, MDIM), lambda i: (0, 0)),
            pl.BlockSpec((SDIM, MDIM), lambda i: (0, 0)),
        ],
        out_specs=(
            pl.BlockSpec((BN, SDIM), lambda i: (i, 0)),
            pl.BlockSpec((BN, V3), lambda i: (i, 0)),
            pl.BlockSpec((BN, MDIM), lambda i: (i, 0)),
            pl.BlockSpec((BN, MDIM), lambda i: (i, 0)),
        ),
        out_shape=out_shapes,
    )(s, vflat, gamma_i.reshape(1, SDIM), beta_i.reshape(1, SDIM), w1, w2)


def _node_post_body(sbar_ref, vbar_ref, sagg_ref, vagg_ref, deg_ref,
                    wu_ref, bu_ref, wv_ref, s_out_ref, v_out_ref, *, last):
    sbar = sbar_ref[...]
    wu = wu_ref[...]
    upd = (jnp.dot(sbar, wu[:SDIM], preferred_element_type=jnp.float32)
           + jnp.dot(sagg_ref[...], wu[SDIM:], preferred_element_type=jnp.float32)
           + bu_ref[...])
    if not last:
        upd = _silu(upd)
    s_out_ref[...] = sbar + upd
    vagg = vagg_ref[...] / deg_ref[...]
    wv = wv_ref[...]
    parts = [jnp.dot(vagg[:, c * VDIM:(c + 1) * VDIM], wv,
                     preferred_element_type=jnp.float32) for c in range(3)]
    v_out_ref[...] = vbar_ref[...] + jnp.concatenate(parts, axis=1)


def _node_post(sbar, vbar, s_agg, v_agg, deg, wu, bu, wv, last):
    n = sbar.shape[0]
    grid = (pl.cdiv(n, BN),)
    out_shapes = (
        jax.ShapeDtypeStruct((n, SDIM), jnp.float32),
        jax.ShapeDtypeStruct((n, V3), jnp.float32),
    )
    return pl.pallas_call(
        functools.partial(_node_post_body, last=last),
        grid=grid,
        in_specs=[
            pl.BlockSpec((BN, SDIM), lambda i: (i, 0)),
            pl.BlockSpec((BN, V3), lambda i: (i, 0)),
            pl.BlockSpec((BN, SDIM), lambda i: (i, 0)),
            pl.BlockSpec((BN, V3), lambda i: (i, 0)),
            pl.BlockSpec((BN, 1), lambda i: (i, 0)),
            pl.BlockSpec((2 * SDIM, SDIM), lambda i: (0, 0)),
            pl.BlockSpec((1, SDIM), lambda i: (0, 0)),
            pl.BlockSpec((VDIM, VDIM), lambda i: (0, 0)),
        ],
        out_specs=(
            pl.BlockSpec((BN, SDIM), lambda i: (i, 0)),
            pl.BlockSpec((BN, V3), lambda i: (i, 0)),
        ),
        out_shape=out_shapes,
    )(sbar, vbar, s_agg, v_agg, deg, wu, bu.reshape(1, SDIM), wv)


def _edge_stage(a_t, b2_t, c_t, w4, r, d, src, dst, edge_attr, vbar, n, first):
    msum = a_t[src] + b2_t[dst] + c_t[edge_attr] + d[:, None] * w4
    m = _silu(msum)
    ms = m[:, :SDIM]
    gv = m[:, SDIM:SDIM + VDIM]
    gr = m[:, SDIM + VDIM:]
    mv = r[:, :, None] * gr[:, None, :]
    if not first:
        e = src.shape[0]
        mv = mv + vbar.reshape(-1, 3, VDIM)[src] * gv[:, None, :]
    s_agg = jax.ops.segment_sum(ms, dst, num_segments=n)
    v_agg = jax.ops.segment_sum(mv.reshape(-1, V3), dst, num_segments=n)
    return s_agg, v_agg


def kernel(x, t, pos, edge_index, edge_attr, batch, atom_emb, bond_emb,
           W_t1, b_t1, W_t2, b_t2, W_msg, b_msg, W_upd, b_upd, W_v,
           gamma, beta, W_down, b_down):
    n = x.shape[0]
    L = W_msg.shape[0]
    src = edge_index[0]
    dst = edge_index[1]

    rvec = pos[dst] - pos[src]
    d = jnp.sqrt(jnp.maximum(jnp.sum(rvec * rvec, axis=-1), 1e-6))
    rvec = rvec / d[:, None]

    temb = _silu(_silu(t @ W_t1 + b_t1) @ W_t2 + b_t2)
    s = atom_emb[x] + temb[batch]
    vflat = jnp.zeros((n, V3), dtype=jnp.float32)

    deg = jnp.maximum(
        jax.ops.segment_sum(jnp.ones_like(d), dst, num_segments=n), 1.0)
    deg2 = deg[:, None]

    for i in range(L):
        w1 = W_msg[i, :SDIM]
        w2 = W_msg[i, SDIM:2 * SDIM]
        w3 = W_msg[i, 2 * SDIM:2 * SDIM + 16]
        w4 = W_msg[i, 2 * SDIM + 16]
        c_t = bond_emb @ w3 + b_msg[i]
        sbar, vbar, a_t, b2_t = _node_pre(s, vflat, gamma[i], beta[i], w1, w2)
        s_agg, v_agg = _edge_stage(a_t, b2_t, c_t, w4, rvec, d, src, dst,
                                   edge_attr, vbar, n, first=(i == 0))
        s, vflat = _node_post(sbar, vbar, s_agg, v_agg, deg2,
                              W_upd[i], b_upd[i], W_v[i], last=(i == L - 1))

    v = vflat.reshape(n, 3, VDIM)
    out = (v @ W_down + b_down).squeeze(-1)
    return out
